# baseline (speedup 1.0000x reference)
import jax
import jax.numpy as jnp
from jax import lax
from jax.experimental import pallas as pl
from jax.experimental.pallas import tpu as pltpu
from jax.experimental.pallas import tpu_sc as plsc

SITU_DIM = 100000
S = 4
D = 128
B = 4096
L = 50

NC, NS = 2, 16
NW = NC * NS
C = B // NW
NBUF = 6
PF = 4
NQ = S * L


def _sc_body(table_hbm, idxr_hbm, o0, o1, o2, o3, idx_v, buf, *sems):
    gsem = sems[:NBUF]
    ssem = sems[NBUF:]
    outs = (o0, o1, o2, o3)
    wid = lax.axis_index("s") * NC + lax.axis_index("c")

    pltpu.sync_copy(idxr_hbm.at[wid], idx_v)

    def g_start(i, l, b):
        pltpu.make_async_copy(
            table_hbm.at[idx_v.at[i, l]], buf.at[b], gsem[b]).start()

    def g_wait(b):
        pltpu.make_async_copy(
            table_hbm.at[idx_v.at[0, 0]], buf.at[b], gsem[b]).wait()

    def s_start(i, l, b):
        pltpu.make_async_copy(
            buf.at[b], outs[i].at[l, pl.ds(wid * C, C)], ssem[b]).start()

    def s_wait(b):
        pltpu.make_async_copy(
            buf.at[b], o0.at[0, pl.ds(0, C)], ssem[b]).wait()

    def step_static(q):
        b = q % NBUF
        g_wait(b)
        s_start(q // L, q % L, b)
        if q >= NBUF - PF:
            s_wait((q - (NBUF - PF)) % NBUF)
        if q + PF < NQ:
            g_start((q + PF) // L, (q + PF) % L, (q + PF) % NBUF)

    for q in range(PF):
        g_start(q // L, q % L, q % NBUF)

    step_static(0)
    step_static(1)
    for i in range(S):
        q0 = i * L + (2 if i == 0 else 0)

        @pl.loop(q0, q0 + 42, step=NBUF)
        def _(g):
            for k in range(NBUF):
                b = (q0 + k) % NBUF
                q = g + k
                g_wait(b)
                s_start(i, q - i * L, b)
                s_wait((b + PF) % NBUF)
                g_start(i, q - i * L + PF, (b + PF) % NBUF)

        for q in range(i * L + (44 if i == 0 else 42), (i + 1) * L):
            step_static(q)

    s_wait((NQ - 2) % NBUF)
    s_wait((NQ - 1) % NBUF)


@jax.jit
def _run(table, idxr):
    out_sds = tuple(
        jax.ShapeDtypeStruct((L, B, D), jnp.float32) for _ in range(S))
    mesh = plsc.VectorSubcoreMesh(core_axis_name="c", subcore_axis_name="s")
    f = pl.kernel(
        _sc_body,
        out_type=out_sds,
        mesh=mesh,
        scratch_types=[
            pltpu.VMEM((S, L, C), jnp.int32),
            pltpu.VMEM((NBUF, C, D), jnp.float32),
        ] + [pltpu.SemaphoreType.DMA] * (2 * NBUF),
        name="emb_gather",
    )
    outs = f(table, idxr)
    return tuple(jnp.transpose(o, (1, 0, 2)) for o in outs)


def kernel(history_context_features, emb_weight):
    idxr = jnp.transpose(
        history_context_features.reshape(NW, C, L, S), (0, 3, 2, 1))
    return _run(emb_weight, idxr)
